# Initial kernel scaffold; baseline (speedup 1.0000x reference)
import jax
import jax.numpy as jnp
from jax.experimental import pallas as pl


def kernel(x, pos_emb, ln_w, ln_b):
    raise NotImplementedError("write your pallas kernel here")


# speedup vs baseline: 2.5537x; 2.5537x over previous
import functools

import jax
import jax.numpy as jnp
from jax.experimental import pallas as pl

SEQ_LEN = 8192
D = 768
B = 2
EPS = 1e-12

ROWS = 512


def _ln_body(x_ref, pe_ref, w_ref, b_ref, o_ref):
    emb = x_ref[0] + pe_ref[...]
    mean = jnp.mean(emb, axis=1, keepdims=True)
    c = emb - mean
    var = jnp.mean(c * c, axis=1, keepdims=True)
    o_ref[0] = c * jax.lax.rsqrt(var + EPS) * w_ref[...] + b_ref[...]


@jax.jit
def kernel(x, pos_emb, ln_w, ln_b):
    w2 = ln_w.reshape(1, D)
    b2 = ln_b.reshape(1, D)
    grid = (B, SEQ_LEN // ROWS)
    return pl.pallas_call(
        _ln_body,
        grid=grid,
        in_specs=[
            pl.BlockSpec((1, ROWS, D), lambda b, i: (b, i, 0)),
            pl.BlockSpec((ROWS, D), lambda b, i: (i, 0)),
            pl.BlockSpec((1, D), lambda b, i: (0, 0)),
            pl.BlockSpec((1, D), lambda b, i: (0, 0)),
        ],
        out_specs=pl.BlockSpec((1, ROWS, D), lambda b, i: (b, i, 0)),
        out_shape=jax.ShapeDtypeStruct((B, SEQ_LEN, D), x.dtype),
    )(x, pos_emb, w2, b2)


# speedup vs baseline: 3.1252x; 1.2238x over previous
import functools

import jax
import jax.numpy as jnp
from jax.experimental import pallas as pl

SEQ_LEN = 8192
D = 768
B = 2
EPS = 1e-12

ROWS = 512


def _ln_body(x_ref, pe_ref, w_ref, b_ref, o_ref):
    emb = x_ref[...] + pe_ref[None]
    mean = jnp.mean(emb, axis=2, keepdims=True)
    c = emb - mean
    var = jnp.mean(c * c, axis=2, keepdims=True)
    o_ref[...] = c * jax.lax.rsqrt(var + EPS) * w_ref[...] + b_ref[...]


@jax.jit
def kernel(x, pos_emb, ln_w, ln_b):
    w2 = ln_w.reshape(1, D)
    b2 = ln_b.reshape(1, D)
    grid = (SEQ_LEN // ROWS,)
    return pl.pallas_call(
        _ln_body,
        grid=grid,
        in_specs=[
            pl.BlockSpec((B, ROWS, D), lambda i: (0, i, 0)),
            pl.BlockSpec((ROWS, D), lambda i: (i, 0)),
            pl.BlockSpec((1, D), lambda i: (0, 0)),
            pl.BlockSpec((1, D), lambda i: (0, 0)),
        ],
        out_specs=pl.BlockSpec((B, ROWS, D), lambda i: (0, i, 0)),
        out_shape=jax.ShapeDtypeStruct((B, SEQ_LEN, D), x.dtype),
    )(x, pos_emb, w2, b2)
